# baseline (speedup 1.0000x reference)
import functools

import jax
import jax.numpy as jnp
from jax.experimental import pallas as pl

_BM = 1024


def _fts_body(h_ref, wt_ref, fts_ref):
    fts = jnp.dot(h_ref[...], wt_ref[0], preferred_element_type=jnp.float32)
    fts_ref[0] = fts.astype(jnp.bfloat16)


def _gcn_body(adj0_ref, adj1_ref, fts_ref, b_ref, a_ref, wfct_ref, bfc_ref,
              e_ref, s_ref):
    i = pl.program_id(1)
    hm = adj0_ref.shape[1]
    col = None
    for k, adj_ref in enumerate((adj0_ref, adj1_ref)):
        adj = adj_ref[0].astype(jnp.bfloat16)
        acc = jnp.dot(adj, fts_ref[0], preferred_element_type=jnp.float32)
        out = acc + b_ref[0]
        out = jnp.where(out >= 0, out, a_ref[0] * out)
        e_ref[0, k * hm:(k + 1) * hm] = out.astype(jnp.bfloat16)
        pre = jnp.dot(out.astype(jnp.bfloat16), wfct_ref[...],
                      preferred_element_type=jnp.float32) + bfc_ref[...]
        part = jnp.sum(jnp.tanh(pre), axis=0, keepdims=True)
        col = part if col is None else col + part

    @pl.when(i == 0)
    def _init():
        s_ref[0] = col

    @pl.when(i > 0)
    def _acc():
        s_ref[0] = s_ref[0] + col


def _combine_body(e_ref, beta_ref, z_ref, *, num_mp):
    z = e_ref[0].astype(jnp.float32) * beta_ref[0]
    for p in range(1, num_mp):
        z = z + e_ref[p].astype(jnp.float32) * beta_ref[p]
    z_ref[...] = z


def _mp_side(h, adj, W, b, a, Wfc, bfc, att):
    P, N, _ = adj.shape
    D = h.shape[1]
    nb = N // _BM

    Wt = jnp.transpose(W, (0, 2, 1))
    fts = pl.pallas_call(
        _fts_body,
        grid=(P,),
        in_specs=[
            pl.BlockSpec((N, D), lambda p: (0, 0)),
            pl.BlockSpec((1, D, D), lambda p: (p, 0, 0)),
        ],
        out_specs=pl.BlockSpec((1, N, D), lambda p: (p, 0, 0)),
        out_shape=jax.ShapeDtypeStruct((P, N, D), jnp.bfloat16),
    )(h, Wt)

    b3 = b.reshape(P, 1, D)
    a3 = jnp.broadcast_to(a.reshape(P, 1, 1), (P, 1, D))
    wfct = Wfc.T.astype(jnp.bfloat16)
    bfc2 = bfc.reshape(1, D)
    e, s = pl.pallas_call(
        _gcn_body,
        grid=(P, nb),
        in_specs=[
            pl.BlockSpec((1, _BM // 2, N), lambda p, i: (p, 2 * i, 0)),
            pl.BlockSpec((1, _BM // 2, N), lambda p, i: (p, 2 * i + 1, 0)),
            pl.BlockSpec((1, N, D), lambda p, i: (p, 0, 0)),
            pl.BlockSpec((1, 1, D), lambda p, i: (p, 0, 0)),
            pl.BlockSpec((1, 1, D), lambda p, i: (p, 0, 0)),
            pl.BlockSpec((D, D), lambda p, i: (0, 0)),
            pl.BlockSpec((1, D), lambda p, i: (0, 0)),
        ],
        out_specs=[
            pl.BlockSpec((1, _BM, D), lambda p, i: (p, i, 0)),
            pl.BlockSpec((1, 1, D), lambda p, i: (p, 0, 0)),
        ],
        out_shape=[
            jax.ShapeDtypeStruct((P, N, D), jnp.bfloat16),
            jax.ShapeDtypeStruct((P, 1, D), jnp.float32),
        ],
    )(adj, adj, fts, b3, a3, wfct, bfc2)

    mean_t = s[:, 0, :] / jnp.float32(N)
    logits = jnp.sum(mean_t * att, axis=1)
    beta = jax.nn.softmax(logits, axis=0)
    beta3 = jnp.broadcast_to(beta.reshape(P, 1, 1), (P, 1, D)).astype(jnp.float32)

    z = pl.pallas_call(
        functools.partial(_combine_body, num_mp=P),
        grid=(nb,),
        in_specs=[
            pl.BlockSpec((P, _BM, D), lambda i: (0, i, 0)),
            pl.BlockSpec((P, 1, D), lambda i: (0, 0, 0)),
        ],
        out_specs=pl.BlockSpec((_BM, D), lambda i: (i, 0)),
        out_shape=jax.ShapeDtypeStruct((N, D), jnp.float32),
    )(e, beta3)
    return z


def kernel(h_d, h_p, mps_d, mps_p, W_dg, b_dg, a_dg, W_pt, b_pt, a_pt,
           Wfc_d, bfc_d, att_d, Wfc_p, bfc_p, att_p):
    z_d = _mp_side(h_d, mps_d, W_dg, b_dg, a_dg, Wfc_d, bfc_d, att_d)
    z_p = _mp_side(h_p, mps_p, W_pt, b_pt, a_pt, Wfc_p, bfc_p, att_p)
    return (z_d, z_p)


# speedup vs baseline: 1.0355x; 1.0355x over previous
import functools

import jax
import jax.numpy as jnp
from jax.experimental import pallas as pl
from jax.experimental.pallas import tpu as pltpu

_BM = 1024


def _gcn_body(adj_ref, h_ref, wt_ref, b_ref, a_ref, wfct_ref, bfc_ref,
              e_ref, s_ref, fts_ref):
    i = pl.program_id(1)

    @pl.when(i == 0)
    def _fts():
        fts = jnp.dot(h_ref[...], wt_ref[0],
                      preferred_element_type=jnp.float32)
        fts_ref[...] = fts.astype(jnp.bfloat16)

    adj = adj_ref[0].astype(jnp.bfloat16)
    acc = jnp.dot(adj, fts_ref[...], preferred_element_type=jnp.float32)
    out = acc + b_ref[0]
    out = jnp.where(out >= 0, out, a_ref[0] * out)
    e_ref[0] = out.astype(jnp.bfloat16)
    pre = jnp.dot(out.astype(jnp.bfloat16), wfct_ref[...],
                  preferred_element_type=jnp.float32) + bfc_ref[...]
    col = jnp.sum(jnp.tanh(pre), axis=0, keepdims=True)

    @pl.when(i == 0)
    def _init():
        s_ref[0] = col

    @pl.when(i > 0)
    def _acc():
        s_ref[0] = s_ref[0] + col


def _combine_body(e_ref, beta_ref, z_ref, *, num_mp):
    z = e_ref[0].astype(jnp.float32) * beta_ref[0]
    for p in range(1, num_mp):
        z = z + e_ref[p].astype(jnp.float32) * beta_ref[p]
    z_ref[...] = z


def _mp_side(h, adj, W, b, a, Wfc, bfc, att):
    P, N, _ = adj.shape
    D = h.shape[1]
    nb = N // _BM

    Wt = jnp.transpose(W, (0, 2, 1))
    b3 = b.reshape(P, 1, D)
    a3 = jnp.broadcast_to(a.reshape(P, 1, 1), (P, 1, D))
    wfct = Wfc.T.astype(jnp.bfloat16)
    bfc2 = bfc.reshape(1, D)
    e, s = pl.pallas_call(
        _gcn_body,
        grid=(P, nb),
        in_specs=[
            pl.BlockSpec((1, _BM, N), lambda p, i: (p, i, 0)),
            pl.BlockSpec((N, D), lambda p, i: (0, 0)),
            pl.BlockSpec((1, D, D), lambda p, i: (p, 0, 0)),
            pl.BlockSpec((1, 1, D), lambda p, i: (p, 0, 0)),
            pl.BlockSpec((1, 1, D), lambda p, i: (p, 0, 0)),
            pl.BlockSpec((D, D), lambda p, i: (0, 0)),
            pl.BlockSpec((1, D), lambda p, i: (0, 0)),
        ],
        out_specs=[
            pl.BlockSpec((1, _BM, D), lambda p, i: (p, i, 0)),
            pl.BlockSpec((1, 1, D), lambda p, i: (p, 0, 0)),
        ],
        out_shape=[
            jax.ShapeDtypeStruct((P, N, D), jnp.bfloat16),
            jax.ShapeDtypeStruct((P, 1, D), jnp.float32),
        ],
        scratch_shapes=[pltpu.VMEM((N, D), jnp.bfloat16)],
    )(adj, h, Wt, b3, a3, wfct, bfc2)

    mean_t = s[:, 0, :] / jnp.float32(N)
    logits = jnp.sum(mean_t * att, axis=1)
    beta = jax.nn.softmax(logits, axis=0)
    beta3 = jnp.broadcast_to(beta.reshape(P, 1, 1), (P, 1, D)).astype(jnp.float32)

    z = pl.pallas_call(
        functools.partial(_combine_body, num_mp=P),
        grid=(nb,),
        in_specs=[
            pl.BlockSpec((P, _BM, D), lambda i: (0, i, 0)),
            pl.BlockSpec((P, 1, D), lambda i: (0, 0, 0)),
        ],
        out_specs=pl.BlockSpec((_BM, D), lambda i: (i, 0)),
        out_shape=jax.ShapeDtypeStruct((N, D), jnp.float32),
    )(e, beta3)
    return z


def kernel(h_d, h_p, mps_d, mps_p, W_dg, b_dg, a_dg, W_pt, b_pt, a_pt,
           Wfc_d, bfc_d, att_d, Wfc_p, bfc_p, att_p):
    z_d = _mp_side(h_d, mps_d, W_dg, b_dg, a_dg, Wfc_d, bfc_d, att_d)
    z_p = _mp_side(h_p, mps_p, W_pt, b_pt, a_pt, Wfc_p, bfc_p, att_p)
    return (z_d, z_p)


# speedup vs baseline: 1.1566x; 1.1169x over previous
import functools

import jax
import jax.numpy as jnp
from jax.experimental import pallas as pl

_BM = 512


def _fts_body(h_d_ref, h_p_ref, wt_ref, fts_ref):
    sdx = pl.program_id(0)

    @pl.when(sdx == 0)
    def _d():
        fts = jnp.dot(h_d_ref[...], wt_ref[0, 0],
                      preferred_element_type=jnp.float32)
        fts_ref[0, 0] = fts.astype(jnp.bfloat16)

    @pl.when(sdx == 1)
    def _p():
        fts = jnp.dot(h_p_ref[...], wt_ref[0, 0],
                      preferred_element_type=jnp.float32)
        fts_ref[0, 0] = fts.astype(jnp.bfloat16)


def _gcn_body(adj_d_ref, adj_p_ref, fts_ref, b_ref, a_ref, wfct_ref,
              bfc_ref, e_ref, s_ref):
    sdx = pl.program_id(0)
    i = pl.program_id(2)

    def work(adj_ref):
        adj = adj_ref[0].astype(jnp.bfloat16)
        acc = jnp.dot(adj, fts_ref[0, 0], preferred_element_type=jnp.float32)
        out = acc + b_ref[0, 0]
        out = jnp.where(out >= 0, out, a_ref[0, 0] * out)
        e_ref[0, 0] = out.astype(jnp.bfloat16)
        pre = jnp.dot(out.astype(jnp.bfloat16), wfct_ref[0],
                      preferred_element_type=jnp.float32) + bfc_ref[0]
        col = jnp.sum(jnp.tanh(pre), axis=0, keepdims=True)

        @pl.when(i == 0)
        def _init():
            s_ref[0, 0] = col

        @pl.when(i > 0)
        def _acc():
            s_ref[0, 0] = s_ref[0, 0] + col

    @pl.when(sdx == 0)
    def _d():
        work(adj_d_ref)

    @pl.when(sdx == 1)
    def _p():
        work(adj_p_ref)


def _combine_body(e_ref, beta_ref, zd_ref, zp_ref, *, num_mp):
    sdx = pl.program_id(0)

    def mix():
        z = e_ref[0, 0].astype(jnp.float32) * beta_ref[0, 0]
        for p in range(1, num_mp):
            z = z + e_ref[0, p].astype(jnp.float32) * beta_ref[0, p]
        return z

    @pl.when(sdx == 0)
    def _d():
        zd_ref[...] = mix()

    @pl.when(sdx == 1)
    def _p():
        zp_ref[...] = mix()


def kernel(h_d, h_p, mps_d, mps_p, W_dg, b_dg, a_dg, W_pt, b_pt, a_pt,
           Wfc_d, bfc_d, att_d, Wfc_p, bfc_p, att_p):
    P, N, _ = mps_d.shape
    D = h_d.shape[1]
    nb = N // _BM

    Wt = jnp.stack([jnp.transpose(W_dg, (0, 2, 1)),
                    jnp.transpose(W_pt, (0, 2, 1))])
    b4 = jnp.stack([b_dg, b_pt]).reshape(2, P, 1, D)
    a4 = jnp.broadcast_to(jnp.stack([a_dg, a_pt]).reshape(2, P, 1, 1),
                          (2, P, 1, D))
    wfct = jnp.stack([Wfc_d.T, Wfc_p.T]).astype(jnp.bfloat16)
    bfc3 = jnp.stack([bfc_d, bfc_p]).reshape(2, 1, D)
    att3 = jnp.stack([att_d, att_p])

    fts = pl.pallas_call(
        _fts_body,
        grid=(2, P),
        in_specs=[
            pl.BlockSpec((N, D), lambda s, p: (0, 0)),
            pl.BlockSpec((N, D), lambda s, p: (0, 0)),
            pl.BlockSpec((1, 1, D, D), lambda s, p: (s, p, 0, 0)),
        ],
        out_specs=pl.BlockSpec((1, 1, N, D), lambda s, p: (s, p, 0, 0)),
        out_shape=jax.ShapeDtypeStruct((2, P, N, D), jnp.bfloat16),
    )(h_d, h_p, Wt)

    e, stat = pl.pallas_call(
        _gcn_body,
        grid=(2, P, nb),
        in_specs=[
            pl.BlockSpec((1, _BM, N),
                         lambda s, p, i: (jnp.where(s == 0, p, P - 1),
                                          jnp.where(s == 0, i, nb - 1), 0)),
            pl.BlockSpec((1, _BM, N),
                         lambda s, p, i: (jnp.where(s == 1, p, 0),
                                          jnp.where(s == 1, i, 0), 0)),
            pl.BlockSpec((1, 1, N, D), lambda s, p, i: (s, p, 0, 0)),
            pl.BlockSpec((1, 1, 1, D), lambda s, p, i: (s, p, 0, 0)),
            pl.BlockSpec((1, 1, 1, D), lambda s, p, i: (s, p, 0, 0)),
            pl.BlockSpec((1, D, D), lambda s, p, i: (s, 0, 0)),
            pl.BlockSpec((1, 1, D), lambda s, p, i: (s, 0, 0)),
        ],
        out_specs=[
            pl.BlockSpec((1, 1, _BM, D), lambda s, p, i: (s, p, i, 0)),
            pl.BlockSpec((1, 1, 1, D), lambda s, p, i: (s, p, 0, 0)),
        ],
        out_shape=[
            jax.ShapeDtypeStruct((2, P, N, D), jnp.bfloat16),
            jax.ShapeDtypeStruct((2, P, 1, D), jnp.float32),
        ],
    )(mps_d, mps_p, fts, b4, a4, wfct, bfc3)

    mean_t = stat[:, :, 0, :] / jnp.float32(N)
    logits = jnp.sum(mean_t * att3, axis=2)
    beta = jax.nn.softmax(logits, axis=1)
    beta4 = jnp.broadcast_to(beta.reshape(2, P, 1, 1),
                             (2, P, 1, D)).astype(jnp.float32)

    z_d, z_p = pl.pallas_call(
        functools.partial(_combine_body, num_mp=P),
        grid=(2, nb),
        in_specs=[
            pl.BlockSpec((1, P, _BM, D), lambda s, i: (s, 0, i, 0)),
            pl.BlockSpec((1, P, 1, D), lambda s, i: (s, 0, 0, 0)),
        ],
        out_specs=[
            pl.BlockSpec((_BM, D),
                         lambda s, i: (jnp.where(s == 0, i, nb - 1), 0)),
            pl.BlockSpec((_BM, D),
                         lambda s, i: (jnp.where(s == 1, i, 0), 0)),
        ],
        out_shape=[
            jax.ShapeDtypeStruct((N, D), jnp.float32),
            jax.ShapeDtypeStruct((N, D), jnp.float32),
        ],
    )(e, beta4)
    return (z_d, z_p)
